# baseline (speedup 1.0000x reference)
import functools

import jax
import jax.numpy as jnp
import numpy as np
from jax import lax
from jax.experimental import pallas as pl
from jax.experimental.pallas import tpu as pltpu
from jax.experimental.pallas import tpu_sc as plsc

N = 10000
E = 320000
D = 128
R = 8

NC = 2
NS = 16
NW = NC * NS
CHUNK = 128
NCHUNK = 80
SB = 8
SBC = NCHUNK // SB
EPT = NCHUNK * CHUNK
EPAD = NW * EPT
TROWS = 640
NACC = NS * TROWS

_PERM = np.zeros((D,), np.int32)
for _q in range(D // 32):
    for _i in range(16):
        _PERM[_q * 32 + _i] = _q * 16 + _i
        _PERM[_q * 32 + 16 + _i] = _q * 16 + _i + 64


def _sc_scatter_body(xt_hbm, src_hbm, rel_hbm, dst_hbm, zeros_hbm, out_hbm,
                     gidx_v, dst_v, praw, rowsf, acc_sh, gsems):
    c = lax.axis_index("c")
    s = lax.axis_index("s")
    wid = s * NC + c

    pltpu.sync_copy(zeros_hbm, acc_sh.at[pl.ds(s * TROWS, TROWS)])
    plsc.subcore_barrier()

    gdummy = xt_hbm.at[pl.ds(0, CHUNK)]

    def wait_g(k):
        pltpu.make_async_copy(gdummy, praw[k], gsems[k]).wait()

    def fire_g(j, k):
        pltpu.async_copy(xt_hbm.at[gidx_v.at[j]], praw[k], gsems[k])

    def convert(k):
        def _cv(e, carry):
            for q in range(D // 32):
                v = praw[k][e, pl.ds(q * 16, 16)]
                ab = plsc.bitcast(v, jnp.bfloat16)
                a, b = plsc.unpack(ab, format=plsc.PackFormat.INTERLEAVED)
                rowsf[e, pl.ds(q * 32, 16)] = a
                rowsf[e, pl.ds(q * 32 + 16, 16)] = b
            return carry

        lax.fori_loop(0, CHUNK, _cv, 0)

    for b in range(SB):
        pltpu.sync_copy(src_hbm.at[wid, b], gidx_v)
        pltpu.sync_copy(rel_hbm.at[wid, b], dst_v)

        def _gidx_chunk(j, carry):
            for k in range(CHUNK // 16):
                sl = pl.ds(k * 16, 16)
                gidx_v[j, sl] = dst_v[j, sl] * N + gidx_v[j, sl]
            return carry

        lax.fori_loop(0, SBC, _gidx_chunk, 0)
        pltpu.sync_copy(dst_hbm.at[wid, b], dst_v)

        fire_g(0, 0)
        fire_g(1, 1)

        def _step(i, carry):
            j = 2 * i
            wait_g(0); convert(0); fire_g(j + 2, 0)
            pltpu.sync_copy(rowsf, acc_sh.at[dst_v.at[j]], add=True)
            wait_g(1); convert(1); fire_g(j + 3, 1)
            pltpu.sync_copy(rowsf, acc_sh.at[dst_v.at[j + 1]], add=True)
            return carry

        lax.fori_loop(0, SBC // 2 - 1, _step, 0)

        wait_g(0); convert(0)
        pltpu.sync_copy(rowsf, acc_sh.at[dst_v.at[SBC - 2]], add=True)
        wait_g(1); convert(1)
        pltpu.sync_copy(rowsf, acc_sh.at[dst_v.at[SBC - 1]], add=True)

    plsc.subcore_barrier()

    @pl.when(c == 0)
    def _():
        pltpu.sync_copy(acc_sh.at[pl.ds(s * TROWS, TROWS)],
                        out_hbm[0].at[pl.ds(s * TROWS, TROWS)])

    @pl.when(c == 1)
    def _():
        pltpu.sync_copy(acc_sh.at[pl.ds(s * TROWS, TROWS)],
                        out_hbm[1].at[pl.ds(s * TROWS, TROWS)])


def _sc_body_wrap(xt_hbm, src_hbm, rel_hbm, dst_hbm, zeros_hbm, out0, out1,
                  gidx_v, dst_v, p0, p1, f0, acc, g0, g1):
    _sc_scatter_body(xt_hbm, src_hbm, rel_hbm, dst_hbm, zeros_hbm,
                     (out0, out1),
                     gidx_v, dst_v, (p0, p1), f0, acc, (g0, g1))


_sc_scatter = pl.kernel(
    _sc_body_wrap,
    out_type=(jax.ShapeDtypeStruct((NACC, D), jnp.float32),
              jax.ShapeDtypeStruct((NACC, D), jnp.float32)),
    mesh=plsc.VectorSubcoreMesh(core_axis_name="c", subcore_axis_name="s"),
    compiler_params=pltpu.CompilerParams(use_tc_tiling_on_sc=False,
                                         needs_layout_passes=False),
    scratch_types=[
        pltpu.VMEM((SBC, CHUNK), jnp.int32),
        pltpu.VMEM((SBC, CHUNK), jnp.int32),
        pltpu.VMEM((CHUNK, D // 2), jnp.int32),
        pltpu.VMEM((CHUNK, D // 2), jnp.int32),
        pltpu.VMEM((CHUNK, D), jnp.float32),
        pltpu.VMEM_SHARED((NACC, D), jnp.float32),
        pltpu.SemaphoreType.DMA,
        pltpu.SemaphoreType.DMA,
    ],
)


def _rxw_body(h_ref, w_ref, o_ref):
    y = jnp.dot(h_ref[...], w_ref[0],
                preferred_element_type=jnp.float32).astype(jnp.bfloat16)
    lo = lax.bitcast_convert_type(y[:, :D // 2], jnp.uint16)
    hi = lax.bitcast_convert_type(y[:, D // 2:], jnp.uint16)
    packed = lo.astype(jnp.uint32) | (hi.astype(jnp.uint32) << 16)
    o_ref[0] = lax.bitcast_convert_type(packed, jnp.int32)


def _rxw(h, w):
    return pl.pallas_call(
        _rxw_body,
        grid=(R,),
        in_specs=[
            pl.BlockSpec((N, D), lambda r: (0, 0)),
            pl.BlockSpec((1, D, D), lambda r: (r, 0, 0)),
        ],
        out_specs=pl.BlockSpec((1, N, D // 2), lambda r: (r, 0, 0)),
        out_shape=jax.ShapeDtypeStruct((R, N, D // 2), jnp.int32),
    )(h, w)


def _add_relu_body(p0_ref, p1_ref, o_ref):
    o_ref[...] = jnp.maximum(p0_ref[...] + p1_ref[...], 0.0)


def _add_relu(p0, p1):
    return pl.pallas_call(
        _add_relu_body,
        grid=(1,),
        in_specs=[
            pl.BlockSpec((N, D), lambda i: (0, 0)),
            pl.BlockSpec((N, D), lambda i: (0, 0)),
        ],
        out_specs=pl.BlockSpec((N, D), lambda i: (0, 0)),
        out_shape=jax.ShapeDtypeStruct((N, D), jnp.float32),
    )(p0, p1)


def _final_body(p0_ref, p1_ref, perm_ref, aw1_ref, ab1_ref, aw2_ref, ab2_ref,
                h_ref, att_ref):
    h2 = jnp.dot(p0_ref[...] + p1_ref[...], perm_ref[...],
                 preferred_element_type=jnp.float32)
    h_ref[...] = h2
    m = jnp.mean(h2, axis=0, keepdims=True)
    a = jnp.maximum(
        jnp.dot(m, aw1_ref[...], preferred_element_type=jnp.float32)
        + ab1_ref[...], 0.0)
    att_ref[...] = jax.nn.sigmoid(
        jnp.dot(a, aw2_ref[...], preferred_element_type=jnp.float32)
        + ab2_ref[...])


def _final(p0, p1, perm_mat, aw1, ab1, aw2, ab2):
    full = lambda a: pl.BlockSpec(a.shape, lambda i: (0,) * a.ndim)
    ab1r = ab1.reshape(1, -1)
    ab2r = ab2.reshape(1, -1)
    return pl.pallas_call(
        _final_body,
        grid=(1,),
        in_specs=[
            pl.BlockSpec((N, D), lambda i: (0, 0)),
            pl.BlockSpec((N, D), lambda i: (0, 0)),
            full(perm_mat), full(aw1), full(ab1r), full(aw2), full(ab2r),
        ],
        out_specs=(
            pl.BlockSpec((N, D), lambda i: (0, 0)),
            pl.BlockSpec((1, 10), lambda i: (0, 0)),
        ),
        out_shape=(
            jax.ShapeDtypeStruct((N, D), jnp.float32),
            jax.ShapeDtypeStruct((1, 10), jnp.float32),
        ),
    )(p0, p1, perm_mat, aw1, ab1r, aw2, ab2r)


def kernel(x, edge_index, edge_type, w1, w2, aw1, ab1, aw2, ab2):
    src = edge_index[0]
    dst = edge_index[1]

    pad = EPAD - E
    srcb = jnp.concatenate([src, jnp.zeros((pad,), jnp.int32)]) \
        .reshape(NW, SB, SBC, CHUNK)
    relb = jnp.concatenate([edge_type, jnp.zeros((pad,), jnp.int32)]) \
        .reshape(NW, SB, SBC, CHUNK)
    pad_dst = N + jnp.arange(pad, dtype=jnp.int32) % (NACC - N)
    dstb = jnp.concatenate([dst, pad_dst]).reshape(NW, SB, SBC, CHUNK)
    ztile = jnp.zeros((TROWS, D), jnp.float32)

    perm = jnp.asarray(_PERM)
    w2_adj = w2[:, perm, :]
    perm_mat = jax.nn.one_hot(perm, D, dtype=jnp.float32)

    def layer(h, w):
        xtp = _rxw(h, w).reshape(R * N, D // 2)
        return _sc_scatter(xtp, srcb, relb, dstb, ztile)

    p0, p1 = layer(x, w1)
    h1 = _add_relu(p0, p1)
    q0, q1 = layer(h1, w2_adj)
    h2, att = _final(q0, q1, perm_mat, aw1, ab1, aw2, ab2)
    return (h2, att)


# speedup vs baseline: 1.1205x; 1.1205x over previous
import functools

import jax
import jax.numpy as jnp
import numpy as np
from jax import lax
from jax.experimental import pallas as pl
from jax.experimental.pallas import tpu as pltpu
from jax.experimental.pallas import tpu_sc as plsc

N = 10000
E = 320000
D = 128
R = 8

NC = 2
NS = 16
NW = NC * NS
CHUNK = 64
NCHUNK = 160
SB = 4
SBC = NCHUNK // SB
EPT = NCHUNK * CHUNK
EPAD = NW * EPT
TROWS = 640
NACC = NS * TROWS

_PERM = np.zeros((D,), np.int32)
for _q in range(D // 32):
    for _i in range(16):
        _PERM[_q * 32 + _i] = _q * 16 + _i
        _PERM[_q * 32 + 16 + _i] = _q * 16 + _i + 64


def _sc_scatter_body(xt_hbm, src_hbm, rel_hbm, dst_hbm, zeros_hbm, out_hbm,
                     gidx_v, dst_v, praw, rowsf, acc_sh, gsems, ssems):
    c = lax.axis_index("c")
    s = lax.axis_index("s")
    wid = s * NC + c

    pltpu.sync_copy(zeros_hbm, acc_sh.at[pl.ds(s * TROWS, TROWS)])
    plsc.subcore_barrier()

    gdummy = xt_hbm.at[pl.ds(0, CHUNK)]
    sdummy = zeros_hbm.at[pl.ds(0, CHUNK)]

    def wait_g(k):
        pltpu.make_async_copy(gdummy, praw[k], gsems[k]).wait()

    def wait_s(k):
        pltpu.make_async_copy(sdummy, rowsf[k], ssems[k]).wait()

    def fire_g(j, k):
        pltpu.async_copy(xt_hbm.at[gidx_v.at[j]], praw[k], gsems[k])

    def fire_s(j, k):
        pltpu.async_copy(rowsf[k], acc_sh.at[dst_v.at[j]], ssems[k], add=True)

    def convert(k):
        def _cv(e, carry):
            for q in range(D // 32):
                v = praw[k][e, pl.ds(q * 16, 16)]
                ab = plsc.bitcast(v, jnp.bfloat16)
                a, b = plsc.unpack(ab, format=plsc.PackFormat.INTERLEAVED)
                rowsf[k][e, pl.ds(q * 32, 16)] = a
                rowsf[k][e, pl.ds(q * 32 + 16, 16)] = b
            return carry

        lax.fori_loop(0, CHUNK, _cv, 0)

    for b in range(SB):
        pltpu.sync_copy(src_hbm.at[wid, b], gidx_v)
        pltpu.sync_copy(rel_hbm.at[wid, b], dst_v)

        def _gidx_chunk(j, carry):
            for k in range(CHUNK // 16):
                sl = pl.ds(k * 16, 16)
                gidx_v[j, sl] = dst_v[j, sl] * N + gidx_v[j, sl]
            return carry

        lax.fori_loop(0, SBC, _gidx_chunk, 0)
        pltpu.sync_copy(dst_hbm.at[wid, b], dst_v)

        fire_g(0, 0)
        fire_g(1, 1)
        wait_g(0); convert(0); fire_s(0, 0); fire_g(2, 0)
        wait_g(1); convert(1); fire_s(1, 1); fire_g(3, 1)

        def _step(i, carry):
            j = 2 * i + 2
            wait_g(0); wait_s(0); convert(0); fire_s(j, 0); fire_g(j + 2, 0)
            wait_g(1); wait_s(1); convert(1); fire_s(j + 1, 1); fire_g(j + 3, 1)
            return carry

        lax.fori_loop(0, (SBC - 4) // 2, _step, 0)

        wait_g(0); wait_s(0); convert(0); fire_s(SBC - 2, 0)
        wait_g(1); wait_s(1); convert(1); fire_s(SBC - 1, 1)
        wait_s(0); wait_s(1)

    plsc.subcore_barrier()

    @pl.when(c == 0)
    def _():
        pltpu.sync_copy(acc_sh.at[pl.ds(s * TROWS, TROWS)],
                        out_hbm[0].at[pl.ds(s * TROWS, TROWS)])

    @pl.when(c == 1)
    def _():
        pltpu.sync_copy(acc_sh.at[pl.ds(s * TROWS, TROWS)],
                        out_hbm[1].at[pl.ds(s * TROWS, TROWS)])


def _sc_body_wrap(xt_hbm, src_hbm, rel_hbm, dst_hbm, zeros_hbm, out0, out1,
                  gidx_v, dst_v, p0, p1, f0, f1, acc, g0, g1, s0, s1):
    _sc_scatter_body(xt_hbm, src_hbm, rel_hbm, dst_hbm, zeros_hbm,
                     (out0, out1),
                     gidx_v, dst_v, (p0, p1), (f0, f1), acc,
                     (g0, g1), (s0, s1))


_sc_scatter = pl.kernel(
    _sc_body_wrap,
    out_type=(jax.ShapeDtypeStruct((NACC, D), jnp.float32),
              jax.ShapeDtypeStruct((NACC, D), jnp.float32)),
    mesh=plsc.VectorSubcoreMesh(core_axis_name="c", subcore_axis_name="s"),
    compiler_params=pltpu.CompilerParams(use_tc_tiling_on_sc=False,
                                         needs_layout_passes=False),
    scratch_types=[
        pltpu.VMEM((SBC, CHUNK), jnp.int32),
        pltpu.VMEM((SBC, CHUNK), jnp.int32),
        pltpu.VMEM((CHUNK, D // 2), jnp.int32),
        pltpu.VMEM((CHUNK, D // 2), jnp.int32),
        pltpu.VMEM((CHUNK, D), jnp.float32),
        pltpu.VMEM((CHUNK, D), jnp.float32),
        pltpu.VMEM_SHARED((NACC, D), jnp.float32),
        pltpu.SemaphoreType.DMA,
        pltpu.SemaphoreType.DMA,
        pltpu.SemaphoreType.DMA,
        pltpu.SemaphoreType.DMA,
    ],
)


def _rxw_body(h_ref, w_ref, o_ref):
    y = jnp.dot(h_ref[...], w_ref[0],
                preferred_element_type=jnp.float32).astype(jnp.bfloat16)
    lo = lax.bitcast_convert_type(y[:, :D // 2], jnp.uint16)
    hi = lax.bitcast_convert_type(y[:, D // 2:], jnp.uint16)
    packed = lo.astype(jnp.uint32) | (hi.astype(jnp.uint32) << 16)
    o_ref[0] = lax.bitcast_convert_type(packed, jnp.int32)


def _rxw(h, w):
    return pl.pallas_call(
        _rxw_body,
        grid=(R,),
        in_specs=[
            pl.BlockSpec((N, D), lambda r: (0, 0)),
            pl.BlockSpec((1, D, D), lambda r: (r, 0, 0)),
        ],
        out_specs=pl.BlockSpec((1, N, D // 2), lambda r: (r, 0, 0)),
        out_shape=jax.ShapeDtypeStruct((R, N, D // 2), jnp.int32),
    )(h, w)


def _add_relu_body(p0_ref, p1_ref, o_ref):
    o_ref[...] = jnp.maximum(p0_ref[...] + p1_ref[...], 0.0)


def _add_relu(p0, p1):
    return pl.pallas_call(
        _add_relu_body,
        grid=(1,),
        in_specs=[
            pl.BlockSpec((N, D), lambda i: (0, 0)),
            pl.BlockSpec((N, D), lambda i: (0, 0)),
        ],
        out_specs=pl.BlockSpec((N, D), lambda i: (0, 0)),
        out_shape=jax.ShapeDtypeStruct((N, D), jnp.float32),
    )(p0, p1)


def _final_body(p0_ref, p1_ref, perm_ref, aw1_ref, ab1_ref, aw2_ref, ab2_ref,
                h_ref, att_ref):
    h2 = jnp.dot(p0_ref[...] + p1_ref[...], perm_ref[...],
                 preferred_element_type=jnp.float32)
    h_ref[...] = h2
    m = jnp.mean(h2, axis=0, keepdims=True)
    a = jnp.maximum(
        jnp.dot(m, aw1_ref[...], preferred_element_type=jnp.float32)
        + ab1_ref[...], 0.0)
    att_ref[...] = jax.nn.sigmoid(
        jnp.dot(a, aw2_ref[...], preferred_element_type=jnp.float32)
        + ab2_ref[...])


def _final(p0, p1, perm_mat, aw1, ab1, aw2, ab2):
    full = lambda a: pl.BlockSpec(a.shape, lambda i: (0,) * a.ndim)
    ab1r = ab1.reshape(1, -1)
    ab2r = ab2.reshape(1, -1)
    return pl.pallas_call(
        _final_body,
        grid=(1,),
        in_specs=[
            pl.BlockSpec((N, D), lambda i: (0, 0)),
            pl.BlockSpec((N, D), lambda i: (0, 0)),
            full(perm_mat), full(aw1), full(ab1r), full(aw2), full(ab2r),
        ],
        out_specs=(
            pl.BlockSpec((N, D), lambda i: (0, 0)),
            pl.BlockSpec((1, 10), lambda i: (0, 0)),
        ),
        out_shape=(
            jax.ShapeDtypeStruct((N, D), jnp.float32),
            jax.ShapeDtypeStruct((1, 10), jnp.float32),
        ),
    )(p0, p1, perm_mat, aw1, ab1r, aw2, ab2r)


def kernel(x, edge_index, edge_type, w1, w2, aw1, ab1, aw2, ab2):
    src = edge_index[0]
    dst = edge_index[1]

    pad = EPAD - E
    srcb = jnp.concatenate([src, jnp.zeros((pad,), jnp.int32)]) \
        .reshape(NW, SB, SBC, CHUNK)
    relb = jnp.concatenate([edge_type, jnp.zeros((pad,), jnp.int32)]) \
        .reshape(NW, SB, SBC, CHUNK)
    pad_dst = N + jnp.arange(pad, dtype=jnp.int32) % (NACC - N)
    dstb = jnp.concatenate([dst, pad_dst]).reshape(NW, SB, SBC, CHUNK)
    ztile = jnp.zeros((TROWS, D), jnp.float32)

    perm = jnp.asarray(_PERM)
    w2_adj = w2[:, perm, :]
    perm_mat = jax.nn.one_hot(perm, D, dtype=jnp.float32)

    def layer(h, w):
        xtp = _rxw(h, w).reshape(R * N, D // 2)
        return _sc_scatter(xtp, srcb, relb, dstb, ztile)

    p0, p1 = layer(x, w1)
    h1 = _add_relu(p0, p1)
    q0, q1 = layer(h1, w2_adj)
    h2, att = _final(q0, q1, perm_mat, aw1, ab1, aw2, ab2)
    return (h2, att)


# speedup vs baseline: 1.1932x; 1.0649x over previous
import functools

import jax
import jax.numpy as jnp
import numpy as np
from jax import lax
from jax.experimental import pallas as pl
from jax.experimental.pallas import tpu as pltpu
from jax.experimental.pallas import tpu_sc as plsc

N = 10000
E = 320000
D = 128
R = 8

NC = 2
NS = 16
NW = NC * NS
CHUNK = 64
NCHUNK = 160
SB = 1
SBC = NCHUNK // SB
EPT = NCHUNK * CHUNK
EPAD = NW * EPT
TROWS = 640
NACC = NS * TROWS

_PERM = np.zeros((D,), np.int32)
for _q in range(D // 32):
    for _i in range(16):
        _PERM[_q * 32 + _i] = _q * 16 + _i
        _PERM[_q * 32 + 16 + _i] = _q * 16 + _i + 64


def _sc_scatter_body(xt_hbm, src_hbm, rel_hbm, dst_hbm, zeros_hbm, out_hbm,
                     gidx_v, dst_v, praw, rowsf, acc_sh, gsems, ssems):
    c = lax.axis_index("c")
    s = lax.axis_index("s")
    wid = s * NC + c

    pltpu.sync_copy(zeros_hbm, acc_sh.at[pl.ds(s * TROWS, TROWS)])
    plsc.subcore_barrier()

    gdummy = xt_hbm.at[pl.ds(0, CHUNK)]
    sdummy = zeros_hbm.at[pl.ds(0, CHUNK)]

    def wait_g(k):
        pltpu.make_async_copy(gdummy, praw[k], gsems[k]).wait()

    def wait_s(k):
        pltpu.make_async_copy(sdummy, rowsf[k], ssems[k]).wait()

    def fire_g(j, k):
        pltpu.async_copy(xt_hbm.at[gidx_v.at[j]], praw[k], gsems[k])

    def fire_s(j, k):
        pltpu.async_copy(rowsf[k], acc_sh.at[dst_v.at[j]], ssems[k], add=True)

    def convert(k):
        def _cv(e, carry):
            for q in range(D // 32):
                v = praw[k][e, pl.ds(q * 16, 16)]
                ab = plsc.bitcast(v, jnp.bfloat16)
                a, b = plsc.unpack(ab, format=plsc.PackFormat.INTERLEAVED)
                rowsf[k][e, pl.ds(q * 32, 16)] = a
                rowsf[k][e, pl.ds(q * 32 + 16, 16)] = b
            return carry

        lax.fori_loop(0, CHUNK, _cv, 0)

    for b in range(SB):
        pltpu.sync_copy(src_hbm.at[wid, b], gidx_v)
        pltpu.sync_copy(rel_hbm.at[wid, b], dst_v)

        def _gidx_chunk(j, carry):
            for k in range(CHUNK // 16):
                sl = pl.ds(k * 16, 16)
                gidx_v[j, sl] = dst_v[j, sl] * N + gidx_v[j, sl]
            return carry

        lax.fori_loop(0, SBC, _gidx_chunk, 0)
        pltpu.sync_copy(dst_hbm.at[wid, b], dst_v)

        fire_g(0, 0)
        fire_g(1, 1)
        wait_g(0); convert(0); fire_s(0, 0); fire_g(2, 0)
        wait_g(1); convert(1); fire_s(1, 1); fire_g(3, 1)

        def _step(i, carry):
            j = 2 * i + 2
            wait_g(0); wait_s(0); convert(0); fire_s(j, 0); fire_g(j + 2, 0)
            wait_g(1); wait_s(1); convert(1); fire_s(j + 1, 1); fire_g(j + 3, 1)
            return carry

        lax.fori_loop(0, (SBC - 4) // 2, _step, 0)

        wait_g(0); wait_s(0); convert(0); fire_s(SBC - 2, 0)
        wait_g(1); wait_s(1); convert(1); fire_s(SBC - 1, 1)
        wait_s(0); wait_s(1)

    plsc.subcore_barrier()

    @pl.when(c == 0)
    def _():
        pltpu.sync_copy(acc_sh.at[pl.ds(s * TROWS, TROWS)],
                        out_hbm[0].at[pl.ds(s * TROWS, TROWS)])

    @pl.when(c == 1)
    def _():
        pltpu.sync_copy(acc_sh.at[pl.ds(s * TROWS, TROWS)],
                        out_hbm[1].at[pl.ds(s * TROWS, TROWS)])


def _sc_body_wrap(xt_hbm, src_hbm, rel_hbm, dst_hbm, zeros_hbm, out0, out1,
                  gidx_v, dst_v, p0, p1, f0, f1, acc, g0, g1, s0, s1):
    _sc_scatter_body(xt_hbm, src_hbm, rel_hbm, dst_hbm, zeros_hbm,
                     (out0, out1),
                     gidx_v, dst_v, (p0, p1), (f0, f1), acc,
                     (g0, g1), (s0, s1))


_sc_scatter = pl.kernel(
    _sc_body_wrap,
    out_type=(jax.ShapeDtypeStruct((NACC, D), jnp.float32),
              jax.ShapeDtypeStruct((NACC, D), jnp.float32)),
    mesh=plsc.VectorSubcoreMesh(core_axis_name="c", subcore_axis_name="s"),
    compiler_params=pltpu.CompilerParams(use_tc_tiling_on_sc=False,
                                         needs_layout_passes=False),
    scratch_types=[
        pltpu.VMEM((SBC, CHUNK), jnp.int32),
        pltpu.VMEM((SBC, CHUNK), jnp.int32),
        pltpu.VMEM((CHUNK, D // 2), jnp.int32),
        pltpu.VMEM((CHUNK, D // 2), jnp.int32),
        pltpu.VMEM((CHUNK, D), jnp.float32),
        pltpu.VMEM((CHUNK, D), jnp.float32),
        pltpu.VMEM_SHARED((NACC, D), jnp.float32),
        pltpu.SemaphoreType.DMA,
        pltpu.SemaphoreType.DMA,
        pltpu.SemaphoreType.DMA,
        pltpu.SemaphoreType.DMA,
    ],
)


def _rxw_body(h_ref, w_ref, o_ref):
    y = jnp.dot(h_ref[...], w_ref[0],
                preferred_element_type=jnp.float32).astype(jnp.bfloat16)
    lo = lax.bitcast_convert_type(y[:, :D // 2], jnp.uint16)
    hi = lax.bitcast_convert_type(y[:, D // 2:], jnp.uint16)
    packed = lo.astype(jnp.uint32) | (hi.astype(jnp.uint32) << 16)
    o_ref[0] = lax.bitcast_convert_type(packed, jnp.int32)


def _rxw(h, w):
    return pl.pallas_call(
        _rxw_body,
        grid=(R,),
        in_specs=[
            pl.BlockSpec((N, D), lambda r: (0, 0)),
            pl.BlockSpec((1, D, D), lambda r: (r, 0, 0)),
        ],
        out_specs=pl.BlockSpec((1, N, D // 2), lambda r: (r, 0, 0)),
        out_shape=jax.ShapeDtypeStruct((R, N, D // 2), jnp.int32),
    )(h, w)


def _add_relu_body(p0_ref, p1_ref, o_ref):
    o_ref[...] = jnp.maximum(p0_ref[...] + p1_ref[...], 0.0)


def _add_relu(p0, p1):
    return pl.pallas_call(
        _add_relu_body,
        grid=(1,),
        in_specs=[
            pl.BlockSpec((N, D), lambda i: (0, 0)),
            pl.BlockSpec((N, D), lambda i: (0, 0)),
        ],
        out_specs=pl.BlockSpec((N, D), lambda i: (0, 0)),
        out_shape=jax.ShapeDtypeStruct((N, D), jnp.float32),
    )(p0, p1)


def _final_body(p0_ref, p1_ref, perm_ref, aw1_ref, ab1_ref, aw2_ref, ab2_ref,
                h_ref, att_ref):
    h2 = jnp.dot(p0_ref[...] + p1_ref[...], perm_ref[...],
                 preferred_element_type=jnp.float32)
    h_ref[...] = h2
    m = jnp.mean(h2, axis=0, keepdims=True)
    a = jnp.maximum(
        jnp.dot(m, aw1_ref[...], preferred_element_type=jnp.float32)
        + ab1_ref[...], 0.0)
    att_ref[...] = jax.nn.sigmoid(
        jnp.dot(a, aw2_ref[...], preferred_element_type=jnp.float32)
        + ab2_ref[...])


def _final(p0, p1, perm_mat, aw1, ab1, aw2, ab2):
    full = lambda a: pl.BlockSpec(a.shape, lambda i: (0,) * a.ndim)
    ab1r = ab1.reshape(1, -1)
    ab2r = ab2.reshape(1, -1)
    return pl.pallas_call(
        _final_body,
        grid=(1,),
        in_specs=[
            pl.BlockSpec((N, D), lambda i: (0, 0)),
            pl.BlockSpec((N, D), lambda i: (0, 0)),
            full(perm_mat), full(aw1), full(ab1r), full(aw2), full(ab2r),
        ],
        out_specs=(
            pl.BlockSpec((N, D), lambda i: (0, 0)),
            pl.BlockSpec((1, 10), lambda i: (0, 0)),
        ),
        out_shape=(
            jax.ShapeDtypeStruct((N, D), jnp.float32),
            jax.ShapeDtypeStruct((1, 10), jnp.float32),
        ),
    )(p0, p1, perm_mat, aw1, ab1r, aw2, ab2r)


def kernel(x, edge_index, edge_type, w1, w2, aw1, ab1, aw2, ab2):
    src = edge_index[0]
    dst = edge_index[1]

    pad = EPAD - E
    srcb = jnp.concatenate([src, jnp.zeros((pad,), jnp.int32)]) \
        .reshape(NW, SB, SBC, CHUNK)
    relb = jnp.concatenate([edge_type, jnp.zeros((pad,), jnp.int32)]) \
        .reshape(NW, SB, SBC, CHUNK)
    pad_dst = N + jnp.arange(pad, dtype=jnp.int32) % (NACC - N)
    dstb = jnp.concatenate([dst, pad_dst]).reshape(NW, SB, SBC, CHUNK)
    ztile = jnp.zeros((TROWS, D), jnp.float32)

    perm = jnp.asarray(_PERM)
    w2_adj = w2[:, perm, :]
    perm_mat = jax.nn.one_hot(perm, D, dtype=jnp.float32)

    def layer(h, w):
        xtp = _rxw(h, w).reshape(R * N, D // 2)
        return _sc_scatter(xtp, srcb, relb, dstb, ztile)

    p0, p1 = layer(x, w1)
    h1 = _add_relu(p0, p1)
    q0, q1 = layer(h1, w2_adj)
    h2, att = _final(q0, q1, perm_mat, aw1, ab1, aw2, ab2)
    return (h2, att)
